# baseline (speedup 1.0000x reference)
import functools

import jax
import jax.numpy as jnp
from jax import lax
from jax.experimental import pallas as pl
from jax.experimental.pallas import tpu as pltpu
from jax.experimental.pallas import tpu_sc as plsc

N = 10000
E = 320000
ND = 128
ED = 16
HD = 256

NW = 32
NT = 16
CHUNK = 128
E_PAD = 327680
E_H = 163840
EH_REAL = E // 2
N_PAD = 10240
BE = 1000
BN = 1024
RW = 4

_f32 = jnp.float32
_bf16 = jnp.bfloat16


def _silu(v):
    return v * jax.nn.sigmoid(v)


def _layernorm(v, g, b):
    m = v.mean(axis=-1, keepdims=True)
    var = v.var(axis=-1, keepdims=True)
    return (v - m) / jnp.sqrt(var + 1e-5) * g + b


def _gather_body(ep, h_ta, h_tb, x0, x1, src, dst, gs_out, gd_out, rec_out,
                 sidx, didx, x0_v, x1_v, srows, drows, rec,
                 isem0, isem1, gsem0, gsem1, osem0, osem1):
    cid = lax.axis_index("c")
    sid = lax.axis_index("s")
    pair = 2 * (ep // NW)
    n0 = (pair // CHUNK) * 3 // 10
    n0 = (n0 // 2) * 2
    n1 = pair // CHUNK - n0
    n_chunks = jnp.where(cid == 0, n0, n1)
    base_w = sid * pair + jnp.where(cid == 0, 0, n0 * CHUNK)

    pltpu.sync_copy(x0, x0_v)
    pltpu.sync_copy(x1, x1_v)
    iota = lax.iota(jnp.int32, 16)
    isems = (isem0, isem1)
    gsems = (gsem0, gsem1)
    osems = (osem0, osem1)

    def ch_sl(c):
        return pl.ds(pl.multiple_of(base_w + c * CHUNK, CHUNK), CHUNK)

    def buf_sl(b):
        return pl.ds(b * CHUNK, CHUNK)

    def load_idx(c, b):
        pltpu.async_copy(src.at[ch_sl(c)], sidx.at[b], isems[b])
        pltpu.async_copy(dst.at[ch_sl(c)], didx.at[b], isems[b])

    def wait_idx(b):
        sl0 = ch_sl(0)
        pltpu.make_async_copy(src.at[sl0], sidx.at[b], isems[b]).wait()
        pltpu.make_async_copy(dst.at[sl0], didx.at[b], isems[b]).wait()

    def geometry(b):
        for g in range(CHUNK // 16):
            gsl = pl.ds(g * 16, 16)
            si = sidx[b, gsl]
            di = didx[b, gsl]
            xs0 = plsc.load_gather(x0_v, [si])
            xs1 = plsc.load_gather(x1_v, [si])
            xd0 = plsc.load_gather(x0_v, [di])
            xd1 = plsc.load_gather(x1_v, [di])
            r0 = xd0 - xs0
            r1 = xd1 - xs1
            d2 = r0 * r0 + r1 * r1
            rows = b * CHUNK + g * 16 + iota
            zero = iota * 0
            plsc.store_scatter(rec, [rows, zero], r0)
            plsc.store_scatter(rec, [rows, zero + 1], r1)
            plsc.store_scatter(rec, [rows, zero + 2], d2)

    def start_out(c, b):
        sl = ch_sl(c)
        pltpu.async_copy(srows.at[buf_sl(b)], gs_out.at[sl], osems[b])
        pltpu.async_copy(drows.at[buf_sl(b)], gd_out.at[sl], osems[b])
        pltpu.async_copy(rec.at[buf_sl(b)], rec_out.at[sl], osems[b])

    def wait_out(b):
        sl0 = ch_sl(0)
        pltpu.make_async_copy(srows.at[buf_sl(b)], gs_out.at[sl0],
                              osems[b]).wait()
        pltpu.make_async_copy(drows.at[buf_sl(b)], gd_out.at[sl0],
                              osems[b]).wait()
        pltpu.make_async_copy(rec.at[buf_sl(b)], rec_out.at[sl0],
                              osems[b]).wait()

    def run(h_t):
        def start_gather(b):
            pltpu.async_copy(h_t.at[sidx.at[b]], srows.at[buf_sl(b)],
                             gsems[b])
            pltpu.async_copy(h_t.at[didx.at[b]], drows.at[buf_sl(b)],
                             gsems[b])

        def wait_gather(b):
            pltpu.make_async_copy(h_t.at[sidx.at[b]], srows.at[buf_sl(b)],
                                  gsems[b]).wait()
            pltpu.make_async_copy(h_t.at[didx.at[b]], drows.at[buf_sl(b)],
                                  gsems[b]).wait()

        load_idx(0, 0)
        wait_idx(0)
        start_gather(0)
        load_idx(1, 1)

        def body(k, carry):
            i = k * 2

            @pl.when(i > 0)
            def _():
                wait_out(1)

            wait_idx(1)
            start_gather(1)
            wait_gather(0)
            geometry(0)
            start_out(i, 0)

            @pl.when(i + 2 < n_chunks)
            def _():
                load_idx(i + 2, 0)

            wait_gather(1)
            geometry(1)
            start_out(i + 1, 1)

            @pl.when(i + 3 < n_chunks)
            def _():
                load_idx(i + 3, 1)

            @pl.when(i + 2 < n_chunks)
            def _():
                wait_out(0)
                wait_idx(0)
                start_gather(0)

            return carry

        lax.fori_loop(0, n_chunks // 2, body, 0)
        wait_out(0)
        wait_out(1)

    @pl.when(cid == 0)
    def _():
        run(h_ta)

    @pl.when(cid == 1)
    def _():
        run(h_tb)


@functools.lru_cache(maxsize=None)
def _gather_call(ep):
    return functools.partial(
        pl.kernel,
        out_type=[
            jax.ShapeDtypeStruct((ep, ND), _f32),
            jax.ShapeDtypeStruct((ep, ND), _f32),
            jax.ShapeDtypeStruct((ep, RW), _f32),
        ],
        mesh=plsc.VectorSubcoreMesh(core_axis_name="c", subcore_axis_name="s"),
        compiler_params=pltpu.CompilerParams(needs_layout_passes=False),
        scratch_types=[
            pltpu.VMEM((2, CHUNK), jnp.int32),
            pltpu.VMEM((2, CHUNK), jnp.int32),
            pltpu.VMEM((N_PAD,), _f32),
            pltpu.VMEM((N_PAD,), _f32),
            pltpu.VMEM((2 * CHUNK, ND), _f32),
            pltpu.VMEM((2 * CHUNK, ND), _f32),
            pltpu.VMEM((2 * CHUNK, RW), _f32),
            pltpu.SemaphoreType.DMA,
            pltpu.SemaphoreType.DMA,
            pltpu.SemaphoreType.DMA,
            pltpu.SemaphoreType.DMA,
            pltpu.SemaphoreType.DMA,
            pltpu.SemaphoreType.DMA,
        ],
    )(functools.partial(_gather_body, ep))


def _scatter_body(ep, dst, msg_lo, msg_hi, init_lo, init_hi,
                  agg_lo, agg_hi,
                  acc_ref, idx, row, lsem0, lsem1, asem0, asem1):
    cid = lax.axis_index("c")
    sid = lax.axis_index("s")
    rows_per_tile = N_PAD // NT
    rbase = pl.multiple_of(sid * rows_per_tile, rows_per_tile)
    rsl = pl.ds(rbase, rows_per_tile)

    @pl.when(cid == 0)
    def _():
        pltpu.sync_copy(init_lo.at[rsl], acc_ref.at[rsl])

    @pl.when(cid == 1)
    def _():
        pltpu.sync_copy(init_hi.at[rsl], acc_ref.at[rsl])

    plsc.subcore_barrier()

    per_tile = ep // NT
    n_chunks = per_tile // CHUNK
    tbase = pl.multiple_of(sid * per_tile, per_tile)
    lsems = (lsem0, lsem1)
    asems = (asem0, asem1)

    def run(msg):
        def ch_sl(c):
            return pl.ds(pl.multiple_of(tbase + c * CHUNK, CHUNK), CHUNK)

        def buf_sl(b):
            return pl.ds(b * CHUNK, CHUNK)

        def start_load(c, b):
            pltpu.async_copy(dst.at[ch_sl(c)], idx.at[b], lsems[b])
            pltpu.async_copy(msg.at[ch_sl(c)], row.at[buf_sl(b)], lsems[b])

        def wait_load(b):
            sl0 = ch_sl(0)
            pltpu.make_async_copy(dst.at[sl0], idx.at[b], lsems[b]).wait()
            pltpu.make_async_copy(msg.at[sl0], row.at[buf_sl(b)],
                                  lsems[b]).wait()

        def start_add(b):
            pltpu.async_copy(row.at[buf_sl(b)], acc_ref.at[idx.at[b]],
                             asems[b], add=True)

        def wait_add(b):
            pltpu.make_async_copy(row.at[buf_sl(b)], acc_ref.at[idx.at[b]],
                                  asems[b]).wait()

        start_load(0, 0)

        def body(k, carry):
            i = k * 2

            @pl.when(i > 0)
            def _():
                wait_add(1)

            start_load(i + 1, 1)
            wait_load(0)
            start_add(0)
            wait_load(1)
            start_add(1)

            @pl.when(i + 2 < n_chunks)
            def _():
                wait_add(0)
                start_load(i + 2, 0)

            return carry

        lax.fori_loop(0, n_chunks // 2, body, 0)
        wait_add(0)
        wait_add(1)

    @pl.when(cid == 0)
    def _():
        run(msg_lo)

    @pl.when(cid == 1)
    def _():
        run(msg_hi)

    plsc.subcore_barrier()

    @pl.when(cid == 0)
    def _():
        pltpu.sync_copy(acc_ref.at[rsl], agg_lo.at[rsl])

    @pl.when(cid == 1)
    def _():
        pltpu.sync_copy(acc_ref.at[rsl], agg_hi.at[rsl])


@functools.lru_cache(maxsize=None)
def _scatter_call(ep):
    return functools.partial(
        pl.kernel,
        out_type=[
            jax.ShapeDtypeStruct((N_PAD, ND), _f32),
            jax.ShapeDtypeStruct((N_PAD, ND), _f32),
        ],
        mesh=plsc.VectorSubcoreMesh(core_axis_name="c", subcore_axis_name="s"),
        compiler_params=pltpu.CompilerParams(needs_layout_passes=False),
        scratch_types=[
            pltpu.VMEM_SHARED((N_PAD, ND), _f32),
            pltpu.VMEM((2, CHUNK), jnp.int32),
            pltpu.VMEM((2 * CHUNK, ND), _f32),
            pltpu.SemaphoreType.DMA,
            pltpu.SemaphoreType.DMA,
            pltpu.SemaphoreType.DMA,
            pltpu.SemaphoreType.DMA,
        ],
    )(functools.partial(_scatter_body, ep))


_CSL = (N_PAD * RW) // NT


def _coord_body(ep, dst, wv, cpair,
                stage_ref, idx, wvb, cacc_v, tmp_v, sum_v, lsem0, lsem1):
    cid = lax.axis_index("c")
    sid = lax.axis_index("s")
    wid = sid * 2 + cid
    per_w = ep // NW
    n_chunks = per_w // CHUNK
    base_w = pl.multiple_of(wid * per_w, per_w)
    iota = lax.iota(jnp.int32, 16)
    z16 = jnp.zeros((16,), _f32)
    lsems = (lsem0, lsem1)

    def start_load(c, b):
        bb = pl.multiple_of(base_w + c * CHUNK, CHUNK)
        pltpu.async_copy(dst.at[pl.ds(bb, CHUNK)], idx.at[b], lsems[b])
        pltpu.async_copy(wv.at[pl.ds(bb, CHUNK)],
                         wvb.at[pl.ds(b * CHUNK, CHUNK)], lsems[b])

    def wait_load(b):
        pltpu.make_async_copy(dst.at[pl.ds(0, CHUNK)], idx.at[b],
                              lsems[b]).wait()
        pltpu.make_async_copy(wv.at[pl.ds(0, CHUNK)],
                              wvb.at[pl.ds(b * CHUNK, CHUNK)],
                              lsems[b]).wait()

    def zbody(i, carry):
        cacc_v[pl.ds(i * 16, 16)] = z16
        return carry

    start_load(0, 0)
    lax.fori_loop(0, (N_PAD * RW) // 16, zbody, 0)

    def process(b):
        for g in range(CHUNK // 16):
            di = idx[b, pl.ds(g * 16, 16)]
            rows = b * CHUNK + g * 16 + iota
            zero = iota * 0
            dbase = di * RW
            for f in range(3):
                val = plsc.load_gather(wvb, [rows, zero + f])
                plsc.addupdate_scatter(cacc_v, [dbase + f], val)

    def body(k, carry):
        i = k * 2
        start_load(i + 1, 1)
        wait_load(0)
        process(0)

        @pl.when(i + 2 < n_chunks)
        def _():
            start_load(i + 2, 0)

        wait_load(1)
        process(1)
        return carry

    lax.fori_loop(0, n_chunks // 2, body, 0)

    pltpu.sync_copy(cacc_v, stage_ref.at[sid])
    plsc.subcore_barrier()

    sbase = pl.multiple_of(sid * _CSL, _CSL)
    ssl = pl.ds(sbase, _CSL)
    pltpu.sync_copy(stage_ref.at[0, ssl], sum_v)
    for t in range(1, NT):
        pltpu.sync_copy(stage_ref.at[t, ssl], tmp_v)

        def abody(k, carry):
            ks = pl.ds(k * 16, 16)
            sum_v[ks] = sum_v[ks] + tmp_v[ks]
            return carry

        lax.fori_loop(0, _CSL // 16, abody, 0)

    pltpu.sync_copy(sum_v, cpair.at[cid, ssl])


@functools.lru_cache(maxsize=None)
def _coord_call(ep):
    return functools.partial(
        pl.kernel,
        out_type=[
            jax.ShapeDtypeStruct((2, N_PAD * RW), _f32),
        ],
        mesh=plsc.VectorSubcoreMesh(core_axis_name="c", subcore_axis_name="s"),
        compiler_params=pltpu.CompilerParams(needs_layout_passes=False),
        scratch_types=[
            pltpu.VMEM_SHARED((NT, N_PAD * RW), _f32),
            pltpu.VMEM((2, CHUNK), jnp.int32),
            pltpu.VMEM((2 * CHUNK, RW), _f32),
            pltpu.VMEM((N_PAD * RW,), _f32),
            pltpu.VMEM((_CSL,), _f32),
            pltpu.VMEM((_CSL,), _f32),
            pltpu.SemaphoreType.DMA,
            pltpu.SemaphoreType.DMA,
        ],
    )(functools.partial(_coord_body, ep))


def _edge_body(gs, gd, rec, ea,
               w1hs, w1hd, w1d, w1ea, b1, w2, b2,
               we1ea, we1m, eb1, ew2, eb2, elng, elnb,
               cw1, cb1, cw2,
               mlo, mhi, eo, wvo):
    hs = gs[...].astype(_bf16)
    hd = gd[...].astype(_bf16)
    ea_v = ea[...]
    ea_b = ea_v.astype(_bf16)
    rec_v = rec[...]
    rel = rec_v[:, :2]
    d2 = rec_v[:, 2:3]
    dist = jnp.sqrt(d2)

    pre = (jnp.dot(hs, w1hs[...], preferred_element_type=_f32)
           + jnp.dot(hd, w1hd[...], preferred_element_type=_f32)
           + dist * w1d[...]
           + jnp.dot(ea_b, w1ea[...], preferred_element_type=_f32)
           + b1[...])
    m1 = _silu(pre).astype(_bf16)
    msg = _silu(jnp.dot(m1, w2[...], preferred_element_type=_f32) + b2[...])
    mlo[...] = msg[:, :ND]
    mhi[...] = msg[:, ND:]
    msg_b = msg.astype(_bf16)

    te = _silu(jnp.dot(ea_b, we1ea[...], preferred_element_type=_f32)
               + jnp.dot(msg_b, we1m[...], preferred_element_type=_f32)
               + eb1[...]).astype(_bf16)
    en = ea_v + jnp.dot(te, ew2[...], preferred_element_type=_f32) + eb2[...]
    eo[...] = _layernorm(en, elng[...], elnb[...])

    c1 = _silu(jnp.dot(msg_b, cw1[...], preferred_element_type=_f32)
               + cb1[...]).astype(_bf16)
    cw = jnp.tanh(jnp.dot(c1, cw2[...], preferred_element_type=_f32))
    rdir = rel / (dist + 1e-8)
    wvec = cw * rdir
    ones = jnp.ones_like(dist)
    zeros = jnp.zeros((wvec.shape[0], RW - 3), _f32)
    wvo[...] = jnp.concatenate([wvec, ones, zeros], axis=-1)


def _full(shape):
    nd = len(shape)
    return pl.BlockSpec(shape, lambda i, _nd=nd: (0,) * _nd)


def _rows(w, blk):
    return pl.BlockSpec((blk, w), lambda i: (i, 0))


_edge_call = pl.pallas_call(
    _edge_body,
    grid=(EH_REAL // BE,),
    in_specs=[
        _rows(ND, BE), _rows(ND, BE), _rows(RW, BE), _rows(ED, BE),
        _full((ND, HD)), _full((ND, HD)), _full((1, HD)), _full((ED, HD)),
        _full((1, HD)), _full((HD, HD)), _full((1, HD)),
        _full((ED, HD)), _full((HD, HD)), _full((1, HD)), _full((HD, ED)),
        _full((1, ED)), _full((1, ED)), _full((1, ED)),
        _full((HD, ND)), _full((1, ND)), _full((ND, 1)),
    ],
    out_specs=[_rows(ND, BE), _rows(ND, BE), _rows(ED, BE), _rows(RW, BE)],
    out_shape=[
        jax.ShapeDtypeStruct((E_H, ND), _f32),
        jax.ShapeDtypeStruct((E_H, ND), _f32),
        jax.ShapeDtypeStruct((EH_REAL, ED), _f32),
        jax.ShapeDtypeStruct((E_H, RW), _f32),
    ],
    compiler_params=pltpu.CompilerParams(
        dimension_semantics=("arbitrary",),
    ),
)


def _node_body(h, alo, ahi, cp, xp,
               wn1h, wn1lo, wn1hi, nb1, wn2, nb2, lng, lnb,
               ho, xo):
    h_v = h[...]
    t = _silu(jnp.dot(h_v, wn1h[...], preferred_element_type=_f32)
              + jnp.dot(alo[...], wn1lo[...], preferred_element_type=_f32)
              + jnp.dot(ahi[...], wn1hi[...], preferred_element_type=_f32)
              + nb1[...])
    hn = h_v + jnp.dot(t, wn2[...], preferred_element_type=_f32) + nb2[...]
    ho[...] = _layernorm(hn, lng[...], lnb[...])

    ca = jnp.sum(cp[...], axis=0)
    cnt = jnp.maximum(ca[:, 2:3], 1.0)
    delta = ca[:, :2] / cnt
    zeros = jnp.zeros((delta.shape[0], RW - 2), _f32)
    xo[...] = xp[...] + jnp.concatenate([delta, zeros], axis=-1)


_node_call = pl.pallas_call(
    _node_body,
    grid=(N_PAD // BN,),
    in_specs=[
        _rows(ND, BN), _rows(ND, BN), _rows(ND, BN),
        pl.BlockSpec((4, BN, RW), lambda i: (0, i, 0)),
        _rows(RW, BN),
        _full((ND, HD)), _full((ND, HD)), _full((ND, HD)), _full((1, HD)),
        _full((HD, ND)), _full((1, ND)), _full((1, ND)), _full((1, ND)),
    ],
    out_specs=[_rows(ND, BN), _rows(RW, BN)],
    out_shape=[
        jax.ShapeDtypeStruct((N_PAD, ND), _f32),
        jax.ShapeDtypeStruct((N_PAD, RW), _f32),
    ],
    compiler_params=pltpu.CompilerParams(
        dimension_semantics=("arbitrary",),
    ),
)


@jax.jit
def kernel(h, x, edge_index, edge_attr,
           msg_w1, msg_b1, msg_w2, msg_b2,
           node_w1, node_b1, node_w2, node_b2,
           edge_w1, edge_b1, edge_w2, edge_b2,
           coord_w1, coord_b1, coord_w2,
           node_ln_g, node_ln_b, edge_ln_g, edge_ln_b):
    src = edge_index[0].astype(jnp.int32)
    dst = edge_index[1].astype(jnp.int32)
    pad_h = E_H - EH_REAL
    zpad = jnp.zeros((pad_h,), jnp.int32)
    npad = jnp.full((pad_h,), N, jnp.int32)
    src_h = [jnp.concatenate([src[k * EH_REAL:(k + 1) * EH_REAL], zpad])
             for k in range(2)]
    dst_h = [jnp.concatenate([dst[k * EH_REAL:(k + 1) * EH_REAL], npad])
             for k in range(2)]

    h_t = jnp.zeros((N_PAD, ND), _f32).at[:N].set(h)
    h_t2 = lax.optimization_barrier(h_t + 0.0)
    x0 = jnp.zeros((N_PAD,), _f32).at[:N].set(x[:, 0])
    x1 = jnp.zeros((N_PAD,), _f32).at[:N].set(x[:, 1])

    gathered = [_gather_call(E_H)(h_t, h_t2, x0, x1, src_h[k], dst_h[k])
                for k in range(2)]

    eweights = (
        msg_w1[:ND].astype(_bf16), msg_w1[ND:2 * ND].astype(_bf16),
        msg_w1[2 * ND:2 * ND + 1], msg_w1[2 * ND + 1:].astype(_bf16),
        msg_b1.reshape(1, HD),
        msg_w2.astype(_bf16), msg_b2.reshape(1, HD),
        edge_w1[:ED].astype(_bf16), edge_w1[ED:].astype(_bf16),
        edge_b1.reshape(1, HD),
        edge_w2.astype(_bf16), edge_b2.reshape(1, ED),
        edge_ln_g.reshape(1, ED), edge_ln_b.reshape(1, ED),
        coord_w1.astype(_bf16), coord_b1.reshape(1, ND),
        coord_w2.astype(_bf16),
    )
    edged = []
    for k in range(2):
        gs, gd, rec = gathered[k]
        ea_k = lax.slice_in_dim(edge_attr, k * EH_REAL, (k + 1) * EH_REAL)
        edged.append(_edge_call(gs, gd, rec, ea_k, *eweights))

    z128 = jnp.zeros((N_PAD, ND), _f32)
    mlo1, mhi1, eo1, wv1 = edged[0]
    mlo2, mhi2, eo2, wv2 = edged[1]
    alo1, ahi1 = _scatter_call(E_H)(dst_h[0], mlo1, mhi1, z128, z128)
    (cpair1,) = _coord_call(E_H)(dst_h[0], wv1)
    agg_lo, agg_hi = _scatter_call(E_H)(dst_h[1], mlo2, mhi2, alo1, ahi1)
    (cpair2,) = _coord_call(E_H)(dst_h[1], wv2)

    h_p = jnp.zeros((N_PAD, ND), _f32).at[:N].set(h)
    x_p = jnp.zeros((N_PAD, RW), _f32).at[:N, :2].set(x)
    cp4 = jnp.concatenate([cpair1, cpair2]).reshape(4, N_PAD, RW)

    hn, xn = _node_call(
        h_p, agg_lo, agg_hi, cp4, x_p,
        node_w1[:ND], node_w1[ND:2 * ND], node_w1[2 * ND:],
        node_b1.reshape(1, HD), node_w2, node_b2.reshape(1, ND),
        node_ln_g.reshape(1, ND), node_ln_b.reshape(1, ND),
    )

    return (hn[:N], xn[:N, :2], jnp.concatenate([eo1, eo2]))


# speedup vs baseline: 1.0343x; 1.0343x over previous
import functools

import jax
import jax.numpy as jnp
from jax import lax
from jax.experimental import pallas as pl
from jax.experimental.pallas import tpu as pltpu
from jax.experimental.pallas import tpu_sc as plsc

N = 10000
E = 320000
ND = 128
ED = 16
HD = 256

NW = 32
NT = 16
CHUNK = 128
E_PAD = 327680
E_H = 163840
EH_REAL = E // 2
N_PAD = 10240
BE = 1000
BN = 1024
RW = 4

_f32 = jnp.float32
_bf16 = jnp.bfloat16


def _silu(v):
    return v * jax.nn.sigmoid(v)


def _layernorm(v, g, b):
    m = v.mean(axis=-1, keepdims=True)
    var = v.var(axis=-1, keepdims=True)
    return (v - m) / jnp.sqrt(var + 1e-5) * g + b


def _gather_body(ep, h_ta, h_tb, x0, x1, src, dst, gs_out, gd_out, rec_out,
                 sidx, didx, x0_v, x1_v, srows, drows, rec,
                 isem0, isem1, gsem0, gsem1, osem0, osem1):
    cid = lax.axis_index("c")
    sid = lax.axis_index("s")
    pair = 2 * (ep // NW)
    n0 = (pair // CHUNK) * 7 // 10
    n0 = (n0 // 2) * 2
    n1 = pair // CHUNK - n0
    n_chunks = jnp.where(cid == 0, n0, n1)
    base_w = sid * pair + jnp.where(cid == 0, 0, n0 * CHUNK)

    pltpu.sync_copy(x0, x0_v)
    pltpu.sync_copy(x1, x1_v)
    iota = lax.iota(jnp.int32, 16)
    isems = (isem0, isem1)
    gsems = (gsem0, gsem1)
    osems = (osem0, osem1)

    def ch_sl(c):
        return pl.ds(pl.multiple_of(base_w + c * CHUNK, CHUNK), CHUNK)

    def buf_sl(b):
        return pl.ds(b * CHUNK, CHUNK)

    def load_idx(c, b):
        pltpu.async_copy(src.at[ch_sl(c)], sidx.at[b], isems[b])
        pltpu.async_copy(dst.at[ch_sl(c)], didx.at[b], isems[b])

    def wait_idx(b):
        sl0 = ch_sl(0)
        pltpu.make_async_copy(src.at[sl0], sidx.at[b], isems[b]).wait()
        pltpu.make_async_copy(dst.at[sl0], didx.at[b], isems[b]).wait()

    def geometry(b):
        for g in range(CHUNK // 16):
            gsl = pl.ds(g * 16, 16)
            si = sidx[b, gsl]
            di = didx[b, gsl]
            xs0 = plsc.load_gather(x0_v, [si])
            xs1 = plsc.load_gather(x1_v, [si])
            xd0 = plsc.load_gather(x0_v, [di])
            xd1 = plsc.load_gather(x1_v, [di])
            r0 = xd0 - xs0
            r1 = xd1 - xs1
            d2 = r0 * r0 + r1 * r1
            rows = b * CHUNK + g * 16 + iota
            zero = iota * 0
            plsc.store_scatter(rec, [rows, zero], r0)
            plsc.store_scatter(rec, [rows, zero + 1], r1)
            plsc.store_scatter(rec, [rows, zero + 2], d2)

    def start_out(c, b):
        sl = ch_sl(c)
        pltpu.async_copy(srows.at[buf_sl(b)], gs_out.at[sl], osems[b])
        pltpu.async_copy(drows.at[buf_sl(b)], gd_out.at[sl], osems[b])
        pltpu.async_copy(rec.at[buf_sl(b)], rec_out.at[sl], osems[b])

    def wait_out(b):
        sl0 = ch_sl(0)
        pltpu.make_async_copy(srows.at[buf_sl(b)], gs_out.at[sl0],
                              osems[b]).wait()
        pltpu.make_async_copy(drows.at[buf_sl(b)], gd_out.at[sl0],
                              osems[b]).wait()
        pltpu.make_async_copy(rec.at[buf_sl(b)], rec_out.at[sl0],
                              osems[b]).wait()

    def run(h_t):
        def start_gather(b):
            pltpu.async_copy(h_t.at[sidx.at[b]], srows.at[buf_sl(b)],
                             gsems[b])
            pltpu.async_copy(h_t.at[didx.at[b]], drows.at[buf_sl(b)],
                             gsems[b])

        def wait_gather(b):
            pltpu.make_async_copy(h_t.at[sidx.at[b]], srows.at[buf_sl(b)],
                                  gsems[b]).wait()
            pltpu.make_async_copy(h_t.at[didx.at[b]], drows.at[buf_sl(b)],
                                  gsems[b]).wait()

        load_idx(0, 0)
        wait_idx(0)
        start_gather(0)
        load_idx(1, 1)

        def body(k, carry):
            i = k * 2

            @pl.when(i > 0)
            def _():
                wait_out(1)

            wait_idx(1)
            start_gather(1)
            wait_gather(0)
            geometry(0)
            start_out(i, 0)

            @pl.when(i + 2 < n_chunks)
            def _():
                load_idx(i + 2, 0)

            wait_gather(1)
            geometry(1)
            start_out(i + 1, 1)

            @pl.when(i + 3 < n_chunks)
            def _():
                load_idx(i + 3, 1)

            @pl.when(i + 2 < n_chunks)
            def _():
                wait_out(0)
                wait_idx(0)
                start_gather(0)

            return carry

        lax.fori_loop(0, n_chunks // 2, body, 0)
        wait_out(0)
        wait_out(1)

    @pl.when(cid == 0)
    def _():
        run(h_ta)

    @pl.when(cid == 1)
    def _():
        run(h_tb)


@functools.lru_cache(maxsize=None)
def _gather_call(ep):
    return functools.partial(
        pl.kernel,
        out_type=[
            jax.ShapeDtypeStruct((ep, ND), _f32),
            jax.ShapeDtypeStruct((ep, ND), _f32),
            jax.ShapeDtypeStruct((ep, RW), _f32),
        ],
        mesh=plsc.VectorSubcoreMesh(core_axis_name="c", subcore_axis_name="s"),
        compiler_params=pltpu.CompilerParams(needs_layout_passes=False),
        scratch_types=[
            pltpu.VMEM((2, CHUNK), jnp.int32),
            pltpu.VMEM((2, CHUNK), jnp.int32),
            pltpu.VMEM((N_PAD,), _f32),
            pltpu.VMEM((N_PAD,), _f32),
            pltpu.VMEM((2 * CHUNK, ND), _f32),
            pltpu.VMEM((2 * CHUNK, ND), _f32),
            pltpu.VMEM((2 * CHUNK, RW), _f32),
            pltpu.SemaphoreType.DMA,
            pltpu.SemaphoreType.DMA,
            pltpu.SemaphoreType.DMA,
            pltpu.SemaphoreType.DMA,
            pltpu.SemaphoreType.DMA,
            pltpu.SemaphoreType.DMA,
        ],
    )(functools.partial(_gather_body, ep))


def _scatter_body(ep, dst, msg_lo, msg_hi, init_lo, init_hi,
                  agg_lo, agg_hi,
                  acc_ref, idx, row, lsem0, lsem1, asem0, asem1):
    cid = lax.axis_index("c")
    sid = lax.axis_index("s")
    rows_per_tile = N_PAD // NT
    rbase = pl.multiple_of(sid * rows_per_tile, rows_per_tile)
    rsl = pl.ds(rbase, rows_per_tile)

    @pl.when(cid == 0)
    def _():
        pltpu.sync_copy(init_lo.at[rsl], acc_ref.at[rsl])

    @pl.when(cid == 1)
    def _():
        pltpu.sync_copy(init_hi.at[rsl], acc_ref.at[rsl])

    plsc.subcore_barrier()

    per_tile = ep // NT
    n_chunks = per_tile // CHUNK
    tbase = pl.multiple_of(sid * per_tile, per_tile)
    lsems = (lsem0, lsem1)
    asems = (asem0, asem1)

    def run(msg):
        def ch_sl(c):
            return pl.ds(pl.multiple_of(tbase + c * CHUNK, CHUNK), CHUNK)

        def buf_sl(b):
            return pl.ds(b * CHUNK, CHUNK)

        def start_load(c, b):
            pltpu.async_copy(dst.at[ch_sl(c)], idx.at[b], lsems[b])
            pltpu.async_copy(msg.at[ch_sl(c)], row.at[buf_sl(b)], lsems[b])

        def wait_load(b):
            sl0 = ch_sl(0)
            pltpu.make_async_copy(dst.at[sl0], idx.at[b], lsems[b]).wait()
            pltpu.make_async_copy(msg.at[sl0], row.at[buf_sl(b)],
                                  lsems[b]).wait()

        def start_add(b):
            pltpu.async_copy(row.at[buf_sl(b)], acc_ref.at[idx.at[b]],
                             asems[b], add=True)

        def wait_add(b):
            pltpu.make_async_copy(row.at[buf_sl(b)], acc_ref.at[idx.at[b]],
                                  asems[b]).wait()

        start_load(0, 0)

        def body(k, carry):
            i = k * 2

            @pl.when(i > 0)
            def _():
                wait_add(1)

            start_load(i + 1, 1)
            wait_load(0)
            start_add(0)
            wait_load(1)
            start_add(1)

            @pl.when(i + 2 < n_chunks)
            def _():
                wait_add(0)
                start_load(i + 2, 0)

            return carry

        lax.fori_loop(0, n_chunks // 2, body, 0)
        wait_add(0)
        wait_add(1)

    @pl.when(cid == 0)
    def _():
        run(msg_lo)

    @pl.when(cid == 1)
    def _():
        run(msg_hi)

    plsc.subcore_barrier()

    @pl.when(cid == 0)
    def _():
        pltpu.sync_copy(acc_ref.at[rsl], agg_lo.at[rsl])

    @pl.when(cid == 1)
    def _():
        pltpu.sync_copy(acc_ref.at[rsl], agg_hi.at[rsl])


@functools.lru_cache(maxsize=None)
def _scatter_call(ep):
    return functools.partial(
        pl.kernel,
        out_type=[
            jax.ShapeDtypeStruct((N_PAD, ND), _f32),
            jax.ShapeDtypeStruct((N_PAD, ND), _f32),
        ],
        mesh=plsc.VectorSubcoreMesh(core_axis_name="c", subcore_axis_name="s"),
        compiler_params=pltpu.CompilerParams(needs_layout_passes=False),
        scratch_types=[
            pltpu.VMEM_SHARED((N_PAD, ND), _f32),
            pltpu.VMEM((2, CHUNK), jnp.int32),
            pltpu.VMEM((2 * CHUNK, ND), _f32),
            pltpu.SemaphoreType.DMA,
            pltpu.SemaphoreType.DMA,
            pltpu.SemaphoreType.DMA,
            pltpu.SemaphoreType.DMA,
        ],
    )(functools.partial(_scatter_body, ep))


_CSL = (N_PAD * RW) // NT


def _coord_body(ep, dst, wv, cpair,
                stage_ref, idx, wvb, cacc_v, tmp_v, sum_v, lsem0, lsem1):
    cid = lax.axis_index("c")
    sid = lax.axis_index("s")
    wid = sid * 2 + cid
    per_w = ep // NW
    n_chunks = per_w // CHUNK
    base_w = pl.multiple_of(wid * per_w, per_w)
    iota = lax.iota(jnp.int32, 16)
    z16 = jnp.zeros((16,), _f32)
    lsems = (lsem0, lsem1)

    def start_load(c, b):
        bb = pl.multiple_of(base_w + c * CHUNK, CHUNK)
        pltpu.async_copy(dst.at[pl.ds(bb, CHUNK)], idx.at[b], lsems[b])
        pltpu.async_copy(wv.at[pl.ds(bb, CHUNK)],
                         wvb.at[pl.ds(b * CHUNK, CHUNK)], lsems[b])

    def wait_load(b):
        pltpu.make_async_copy(dst.at[pl.ds(0, CHUNK)], idx.at[b],
                              lsems[b]).wait()
        pltpu.make_async_copy(wv.at[pl.ds(0, CHUNK)],
                              wvb.at[pl.ds(b * CHUNK, CHUNK)],
                              lsems[b]).wait()

    def zbody(i, carry):
        cacc_v[pl.ds(i * 16, 16)] = z16
        return carry

    start_load(0, 0)
    lax.fori_loop(0, (N_PAD * RW) // 16, zbody, 0)

    def process(b):
        for g in range(CHUNK // 16):
            di = idx[b, pl.ds(g * 16, 16)]
            rows = b * CHUNK + g * 16 + iota
            zero = iota * 0
            dbase = di * RW
            for f in range(3):
                val = plsc.load_gather(wvb, [rows, zero + f])
                plsc.addupdate_scatter(cacc_v, [dbase + f], val)

    def body(k, carry):
        i = k * 2
        start_load(i + 1, 1)
        wait_load(0)
        process(0)

        @pl.when(i + 2 < n_chunks)
        def _():
            start_load(i + 2, 0)

        wait_load(1)
        process(1)
        return carry

    lax.fori_loop(0, n_chunks // 2, body, 0)

    pltpu.sync_copy(cacc_v, stage_ref.at[sid])
    plsc.subcore_barrier()

    sbase = pl.multiple_of(sid * _CSL, _CSL)
    ssl = pl.ds(sbase, _CSL)
    pltpu.sync_copy(stage_ref.at[0, ssl], sum_v)
    for t in range(1, NT):
        pltpu.sync_copy(stage_ref.at[t, ssl], tmp_v)

        def abody(k, carry):
            ks = pl.ds(k * 16, 16)
            sum_v[ks] = sum_v[ks] + tmp_v[ks]
            return carry

        lax.fori_loop(0, _CSL // 16, abody, 0)

    pltpu.sync_copy(sum_v, cpair.at[cid, ssl])


@functools.lru_cache(maxsize=None)
def _coord_call(ep):
    return functools.partial(
        pl.kernel,
        out_type=[
            jax.ShapeDtypeStruct((2, N_PAD * RW), _f32),
        ],
        mesh=plsc.VectorSubcoreMesh(core_axis_name="c", subcore_axis_name="s"),
        compiler_params=pltpu.CompilerParams(needs_layout_passes=False),
        scratch_types=[
            pltpu.VMEM_SHARED((NT, N_PAD * RW), _f32),
            pltpu.VMEM((2, CHUNK), jnp.int32),
            pltpu.VMEM((2 * CHUNK, RW), _f32),
            pltpu.VMEM((N_PAD * RW,), _f32),
            pltpu.VMEM((_CSL,), _f32),
            pltpu.VMEM((_CSL,), _f32),
            pltpu.SemaphoreType.DMA,
            pltpu.SemaphoreType.DMA,
        ],
    )(functools.partial(_coord_body, ep))


def _edge_body(gs, gd, rec, ea,
               w1hs, w1hd, w1d, w1ea, b1, w2, b2,
               we1ea, we1m, eb1, ew2, eb2, elng, elnb,
               cw1, cb1, cw2,
               mlo, mhi, eo, wvo):
    hs = gs[...].astype(_bf16)
    hd = gd[...].astype(_bf16)
    ea_v = ea[...]
    ea_b = ea_v.astype(_bf16)
    rec_v = rec[...]
    rel = rec_v[:, :2]
    d2 = rec_v[:, 2:3]
    dist = jnp.sqrt(d2)

    pre = (jnp.dot(hs, w1hs[...], preferred_element_type=_f32)
           + jnp.dot(hd, w1hd[...], preferred_element_type=_f32)
           + dist * w1d[...]
           + jnp.dot(ea_b, w1ea[...], preferred_element_type=_f32)
           + b1[...])
    m1 = _silu(pre).astype(_bf16)
    msg = _silu(jnp.dot(m1, w2[...], preferred_element_type=_f32) + b2[...])
    mlo[...] = msg[:, :ND]
    mhi[...] = msg[:, ND:]
    msg_b = msg.astype(_bf16)

    te = _silu(jnp.dot(ea_b, we1ea[...], preferred_element_type=_f32)
               + jnp.dot(msg_b, we1m[...], preferred_element_type=_f32)
               + eb1[...]).astype(_bf16)
    en = ea_v + jnp.dot(te, ew2[...], preferred_element_type=_f32) + eb2[...]
    eo[...] = _layernorm(en, elng[...], elnb[...])

    c1 = _silu(jnp.dot(msg_b, cw1[...], preferred_element_type=_f32)
               + cb1[...]).astype(_bf16)
    cw = jnp.tanh(jnp.dot(c1, cw2[...], preferred_element_type=_f32))
    rdir = rel / (dist + 1e-8)
    wvec = cw * rdir
    ones = jnp.ones_like(dist)
    zeros = jnp.zeros((wvec.shape[0], RW - 3), _f32)
    wvo[...] = jnp.concatenate([wvec, ones, zeros], axis=-1)


def _full(shape):
    nd = len(shape)
    return pl.BlockSpec(shape, lambda i, _nd=nd: (0,) * _nd)


def _rows(w, blk):
    return pl.BlockSpec((blk, w), lambda i: (i, 0))


_edge_call = pl.pallas_call(
    _edge_body,
    grid=(EH_REAL // BE,),
    in_specs=[
        _rows(ND, BE), _rows(ND, BE), _rows(RW, BE), _rows(ED, BE),
        _full((ND, HD)), _full((ND, HD)), _full((1, HD)), _full((ED, HD)),
        _full((1, HD)), _full((HD, HD)), _full((1, HD)),
        _full((ED, HD)), _full((HD, HD)), _full((1, HD)), _full((HD, ED)),
        _full((1, ED)), _full((1, ED)), _full((1, ED)),
        _full((HD, ND)), _full((1, ND)), _full((ND, 1)),
    ],
    out_specs=[_rows(ND, BE), _rows(ND, BE), _rows(ED, BE), _rows(RW, BE)],
    out_shape=[
        jax.ShapeDtypeStruct((E_H, ND), _f32),
        jax.ShapeDtypeStruct((E_H, ND), _f32),
        jax.ShapeDtypeStruct((EH_REAL, ED), _f32),
        jax.ShapeDtypeStruct((E_H, RW), _f32),
    ],
    compiler_params=pltpu.CompilerParams(
        dimension_semantics=("arbitrary",),
    ),
)


def _node_body(h, alo, ahi, cp, xp,
               wn1h, wn1lo, wn1hi, nb1, wn2, nb2, lng, lnb,
               ho, xo):
    h_v = h[...]
    t = _silu(jnp.dot(h_v, wn1h[...], preferred_element_type=_f32)
              + jnp.dot(alo[...], wn1lo[...], preferred_element_type=_f32)
              + jnp.dot(ahi[...], wn1hi[...], preferred_element_type=_f32)
              + nb1[...])
    hn = h_v + jnp.dot(t, wn2[...], preferred_element_type=_f32) + nb2[...]
    ho[...] = _layernorm(hn, lng[...], lnb[...])

    ca = jnp.sum(cp[...], axis=0)
    cnt = jnp.maximum(ca[:, 2:3], 1.0)
    delta = ca[:, :2] / cnt
    zeros = jnp.zeros((delta.shape[0], RW - 2), _f32)
    xo[...] = xp[...] + jnp.concatenate([delta, zeros], axis=-1)


_node_call = pl.pallas_call(
    _node_body,
    grid=(N_PAD // BN,),
    in_specs=[
        _rows(ND, BN), _rows(ND, BN), _rows(ND, BN),
        pl.BlockSpec((4, BN, RW), lambda i: (0, i, 0)),
        _rows(RW, BN),
        _full((ND, HD)), _full((ND, HD)), _full((ND, HD)), _full((1, HD)),
        _full((HD, ND)), _full((1, ND)), _full((1, ND)), _full((1, ND)),
    ],
    out_specs=[_rows(ND, BN), _rows(RW, BN)],
    out_shape=[
        jax.ShapeDtypeStruct((N_PAD, ND), _f32),
        jax.ShapeDtypeStruct((N_PAD, RW), _f32),
    ],
    compiler_params=pltpu.CompilerParams(
        dimension_semantics=("arbitrary",),
    ),
)


@jax.jit
def kernel(h, x, edge_index, edge_attr,
           msg_w1, msg_b1, msg_w2, msg_b2,
           node_w1, node_b1, node_w2, node_b2,
           edge_w1, edge_b1, edge_w2, edge_b2,
           coord_w1, coord_b1, coord_w2,
           node_ln_g, node_ln_b, edge_ln_g, edge_ln_b):
    src = edge_index[0].astype(jnp.int32)
    dst = edge_index[1].astype(jnp.int32)
    pad_h = E_H - EH_REAL
    zpad = jnp.zeros((pad_h,), jnp.int32)
    npad = jnp.full((pad_h,), N, jnp.int32)
    src_h = [jnp.concatenate([src[k * EH_REAL:(k + 1) * EH_REAL], zpad])
             for k in range(2)]
    dst_h = [jnp.concatenate([dst[k * EH_REAL:(k + 1) * EH_REAL], npad])
             for k in range(2)]

    h_t = jnp.zeros((N_PAD, ND), _f32).at[:N].set(h)
    h_t2 = lax.optimization_barrier(h_t + 0.0)
    x0 = jnp.zeros((N_PAD,), _f32).at[:N].set(x[:, 0])
    x1 = jnp.zeros((N_PAD,), _f32).at[:N].set(x[:, 1])

    gathered = [_gather_call(E_H)(h_t, h_t2, x0, x1, src_h[k], dst_h[k])
                for k in range(2)]

    eweights = (
        msg_w1[:ND].astype(_bf16), msg_w1[ND:2 * ND].astype(_bf16),
        msg_w1[2 * ND:2 * ND + 1], msg_w1[2 * ND + 1:].astype(_bf16),
        msg_b1.reshape(1, HD),
        msg_w2.astype(_bf16), msg_b2.reshape(1, HD),
        edge_w1[:ED].astype(_bf16), edge_w1[ED:].astype(_bf16),
        edge_b1.reshape(1, HD),
        edge_w2.astype(_bf16), edge_b2.reshape(1, ED),
        edge_ln_g.reshape(1, ED), edge_ln_b.reshape(1, ED),
        coord_w1.astype(_bf16), coord_b1.reshape(1, ND),
        coord_w2.astype(_bf16),
    )
    edged = []
    for k in range(2):
        gs, gd, rec = gathered[k]
        ea_k = lax.slice_in_dim(edge_attr, k * EH_REAL, (k + 1) * EH_REAL)
        edged.append(_edge_call(gs, gd, rec, ea_k, *eweights))

    z128 = jnp.zeros((N_PAD, ND), _f32)
    mlo1, mhi1, eo1, wv1 = edged[0]
    mlo2, mhi2, eo2, wv2 = edged[1]
    alo1, ahi1 = _scatter_call(E_H)(dst_h[0], mlo1, mhi1, z128, z128)
    (cpair1,) = _coord_call(E_H)(dst_h[0], wv1)
    agg_lo, agg_hi = _scatter_call(E_H)(dst_h[1], mlo2, mhi2, alo1, ahi1)
    (cpair2,) = _coord_call(E_H)(dst_h[1], wv2)

    h_p = jnp.zeros((N_PAD, ND), _f32).at[:N].set(h)
    x_p = jnp.zeros((N_PAD, RW), _f32).at[:N, :2].set(x)
    cp4 = jnp.concatenate([cpair1, cpair2]).reshape(4, N_PAD, RW)

    hn, xn = _node_call(
        h_p, agg_lo, agg_hi, cp4, x_p,
        node_w1[:ND], node_w1[ND:2 * ND], node_w1[2 * ND:],
        node_b1.reshape(1, HD), node_w2, node_b2.reshape(1, ND),
        node_ln_g.reshape(1, ND), node_ln_b.reshape(1, ND),
    )

    return (hn[:N], xn[:N, :2], jnp.concatenate([eo1, eo2]))
